# baseline (speedup 1.0000x reference)
import jax
import jax.numpy as jnp
from jax.experimental import pallas as pl
from jax.experimental.pallas import tpu as pltpu

B, S, H, E, K, I = 1, 2048, 768, 8, 2, 3072
BALANCE_COEF = 0.01
N = B * S * K
TM = 512
NB = N // TM
G = NB + E - 1
J = 4
TI = I // J
LANES = 128


def _router_body(x_ref, wg_ref, idx_ref, w_ref, counts_ref, loss_ref):
    x = x_ref[...]
    wg = wg_ref[...]
    logits = jax.lax.dot_general(
        x, wg, (((1,), (0,)), ((), ())), preferred_element_type=jnp.float32)
    lane = jax.lax.broadcasted_iota(jnp.int32, (S, LANES), 1)
    valid = lane < E
    lg = jnp.where(valid, logits, -1e30)
    m = jnp.max(lg, axis=1, keepdims=True)
    p = jnp.where(valid, jnp.exp(lg - m), 0.0)
    probs = p / jnp.sum(p, axis=1, keepdims=True)
    v1 = jnp.max(probs, axis=1, keepdims=True)
    i1 = jnp.min(jnp.where(probs == v1, lane, LANES), axis=1, keepdims=True)
    probs_m = jnp.where(lane == i1, -1.0, probs)
    v2 = jnp.max(probs_m, axis=1, keepdims=True)
    i2 = jnp.min(jnp.where(probs_m == v2, lane, LANES), axis=1, keepdims=True)
    denom = v1 + v2
    idx_ref[...] = jnp.where(lane == 0, i1,
                             jnp.where(lane == 1, i2, 0)).astype(jnp.int32)
    w_ref[...] = jnp.where(lane == 0, v1 / denom,
                           jnp.where(lane == 1, v2 / denom, 0.0))
    onehot = ((lane == i1) | (lane == i2)).astype(jnp.float32)
    counts = jnp.sum(onehot, axis=0, keepdims=True)
    counts_ref[...] = counts.astype(jnp.int32)
    pmean = jnp.mean(probs, axis=0, keepdims=True)
    f = counts / jnp.float32(S)
    loss = BALANCE_COEF * E * jnp.sum(f * pmean)
    lane0 = jax.lax.broadcasted_iota(jnp.int32, (1, LANES), 1)
    loss_ref[...] = jnp.where(lane0 == 0, loss, 0.0)


def _router(x, wg_padded):
    return pl.pallas_call(
        _router_body,
        out_shape=(
            jax.ShapeDtypeStruct((S, LANES), jnp.int32),
            jax.ShapeDtypeStruct((S, LANES), jnp.float32),
            jax.ShapeDtypeStruct((1, LANES), jnp.int32),
            jax.ShapeDtypeStruct((1, LANES), jnp.float32),
        ),
    )(x, wg_padded)


def _ffn_body(cuts_ref, blk_ref, exp_ref, isf_ref, isl_ref,
              x_ref, *rest):
    w1_refs = rest[:J]
    b1_ref = rest[J]
    w2_refs = rest[J + 1:2 * J + 1]
    b2_ref = rest[2 * J + 1]
    ws_ref = rest[2 * J + 2]
    out_ref = rest[2 * J + 3]
    acc_ref = rest[2 * J + 4]
    g = pl.program_id(0)

    @pl.when(isf_ref[g] == 1)
    def _():
        acc_ref[...] = jnp.zeros_like(acc_ref)

    x = x_ref[...].astype(jnp.bfloat16)
    y = b2_ref[0]
    for j in range(J):
        hj = jnp.dot(x, w1_refs[j][0].astype(jnp.bfloat16),
                     preferred_element_type=jnp.float32)
        hj = hj + b1_ref[0, :, pl.ds(j * TI, TI)]
        hj = jax.nn.gelu(hj).astype(jnp.bfloat16)
        y = y + jnp.dot(hj, w2_refs[j][0].astype(jnp.bfloat16),
                        preferred_element_type=jnp.float32)
    row = blk_ref[g] * TM + jax.lax.broadcasted_iota(jnp.int32, (TM, 1), 0)
    mask = (row >= cuts_ref[g]) & (row < cuts_ref[g + 1])
    mw = jnp.where(mask, ws_ref[...], 0.0)
    acc_ref[...] += mw * y

    @pl.when(isl_ref[g] == 1)
    def _():
        out_ref[...] = acc_ref[...]


def _grouped_ffn(cuts, blk_ids, exp_ids, isf, isl, x_sorted, W1, b1r, W2, b2r, ws2d):
    w1_specs = [
        pl.BlockSpec((1, H, TI), lambda g, c, b, e, f, l, j=j: (e[g], 0, j))
        for j in range(J)
    ]
    w2_specs = [
        pl.BlockSpec((1, TI, H), lambda g, c, b, e, f, l, j=j: (e[g], j, 0))
        for j in range(J)
    ]
    grid_spec = pltpu.PrefetchScalarGridSpec(
        num_scalar_prefetch=5,
        grid=(G,),
        in_specs=[
            pl.BlockSpec((TM, H), lambda g, c, b, e, f, l: (b[g], 0)),
            *w1_specs,
            pl.BlockSpec((1, 1, I), lambda g, c, b, e, f, l: (e[g], 0, 0)),
            *w2_specs,
            pl.BlockSpec((1, 1, H), lambda g, c, b, e, f, l: (e[g], 0, 0)),
            pl.BlockSpec((TM, 1), lambda g, c, b, e, f, l: (b[g], 0)),
        ],
        out_specs=pl.BlockSpec((TM, H), lambda g, c, b, e, f, l: (b[g], 0)),
        scratch_shapes=[pltpu.VMEM((TM, H), jnp.float32)],
    )
    return pl.pallas_call(
        _ffn_body,
        grid_spec=grid_spec,
        out_shape=jax.ShapeDtypeStruct((N, H), jnp.float32),
    )(cuts, blk_ids, exp_ids, isf, isl,
      x_sorted, *([W1] * J), b1r, *([W2] * J), b2r, ws2d)


def kernel(hidden_states, Wg, W1, b1, W2, b2):
    x = hidden_states.reshape(S, H)
    wg_padded = jnp.pad(Wg, ((0, 0), (0, LANES - E)))

    idx_out, w_out, counts_out, loss_out = _router(x, wg_padded)
    balance_loss = loss_out[0, 0]
    counts = counts_out[0, :E]
    experts_flat = idx_out[:, :K].reshape(-1)
    weights_flat = w_out[:, :K].reshape(-1)

    offs = jnp.concatenate([jnp.zeros((1,), jnp.int32),
                            jnp.cumsum(counts, dtype=jnp.int32)])
    sort_idx = jnp.argsort(experts_flat).astype(jnp.int32)
    tok_sorted0 = sort_idx // K
    x_sorted = jnp.take(x, tok_sorted0, axis=0)
    ws2d = jnp.take(weights_flat, sort_idx)[:, None]

    blk_bounds = jnp.arange(NB, dtype=jnp.int32) * TM
    cuts = jnp.sort(jnp.concatenate([blk_bounds, offs[1:E]]))
    cuts_full = jnp.concatenate([cuts, jnp.full((1,), N, jnp.int32)])
    blk_ids = jnp.clip(cuts // TM, 0, NB - 1).astype(jnp.int32)
    exp_ids = jnp.clip(jnp.searchsorted(offs, cuts, side="right") - 1,
                       0, E - 1).astype(jnp.int32)
    prev = jnp.concatenate([jnp.full((1,), -1, jnp.int32), blk_ids[:-1]])
    nxt = jnp.concatenate([blk_ids[1:], jnp.full((1,), -1, jnp.int32)])
    isf = (blk_ids != prev).astype(jnp.int32)
    isl = (blk_ids != nxt).astype(jnp.int32)

    b1r = b1[:, None, :]
    b2r = b2[:, None, :]
    y_sorted = _grouped_ffn(cuts_full, blk_ids, exp_ids, isf, isl,
                            x_sorted, W1, b1r, W2, b2r, ws2d)

    tok_sorted = tok_sorted0
    combined = jnp.zeros((S, H), jnp.float32).at[tok_sorted].add(y_sorted)
    out = (x + combined).reshape(B, S, H)
    return out, balance_loss


# speedup vs baseline: 1.1336x; 1.1336x over previous
import functools

import jax
import jax.numpy as jnp
from jax.experimental import pallas as pl
from jax.experimental.pallas import tpu as pltpu

B, S, H, E, K, I = 1, 2048, 768, 8, 2, 3072
BALANCE_COEF = 0.01
N = B * S * K
TM = 512
NB = N // TM
G = NB + E - 1
LANES = 128


def _router_body(x_ref, wg_ref, idx_ref, w_ref, counts_ref, loss_ref):
    x = x_ref[...]
    wg = wg_ref[...]
    logits = jax.lax.dot_general(
        x, wg, (((1,), (0,)), ((), ())), preferred_element_type=jnp.float32)
    lane = jax.lax.broadcasted_iota(jnp.int32, (S, LANES), 1)
    valid = lane < E
    lg = jnp.where(valid, logits, -1e30)
    m = jnp.max(lg, axis=1, keepdims=True)
    p = jnp.where(valid, jnp.exp(lg - m), 0.0)
    probs = p / jnp.sum(p, axis=1, keepdims=True)
    v1 = jnp.max(probs, axis=1, keepdims=True)
    i1 = jnp.min(jnp.where(probs == v1, lane, LANES), axis=1, keepdims=True)
    probs_m = jnp.where(lane == i1, -1.0, probs)
    v2 = jnp.max(probs_m, axis=1, keepdims=True)
    i2 = jnp.min(jnp.where(probs_m == v2, lane, LANES), axis=1, keepdims=True)
    denom = v1 + v2
    idx_ref[...] = jnp.where(lane == 0, i1,
                             jnp.where(lane == 1, i2, 0)).astype(jnp.int32)
    w_ref[...] = jnp.where(lane == 0, v1 / denom,
                           jnp.where(lane == 1, v2 / denom, 0.0))
    onehot = ((lane == i1) | (lane == i2)).astype(jnp.float32)
    counts = jnp.sum(onehot, axis=0, keepdims=True)
    counts_ref[...] = counts.astype(jnp.int32)
    pmean = jnp.mean(probs, axis=0, keepdims=True)
    f = counts / jnp.float32(S)
    loss = BALANCE_COEF * E * jnp.sum(f * pmean)
    lane0 = jax.lax.broadcasted_iota(jnp.int32, (1, LANES), 1)
    loss_ref[...] = jnp.where(lane0 == 0, loss, 0.0)


def _router(x, wg_padded):
    return pl.pallas_call(
        _router_body,
        out_shape=(
            jax.ShapeDtypeStruct((S, LANES), jnp.int32),
            jax.ShapeDtypeStruct((S, LANES), jnp.float32),
            jax.ShapeDtypeStruct((1, LANES), jnp.int32),
            jax.ShapeDtypeStruct((1, LANES), jnp.float32),
        ),
    )(x, wg_padded)


def _ffn_body(cuts_ref, blk_ref, exp_ref, isf_ref, isl_ref,
              x_ref, w1_ref, b1_ref, w2_ref, b2_ref, ws_ref, out_ref):
    g = pl.program_id(0)

    @pl.when(isf_ref[g] == 1)
    def _():
        out_ref[...] = jnp.zeros_like(out_ref)

    x = x_ref[...].astype(jnp.bfloat16)
    h = jnp.dot(x, w1_ref[0].astype(jnp.bfloat16),
                preferred_element_type=jnp.float32) + b1_ref[0]
    h = jax.nn.gelu(h).astype(jnp.bfloat16)
    y = jnp.dot(h, w2_ref[0].astype(jnp.bfloat16),
                preferred_element_type=jnp.float32) + b2_ref[0]
    row = blk_ref[g] * TM + jax.lax.broadcasted_iota(jnp.int32, (TM, 1), 0)
    mask = (row >= cuts_ref[g]) & (row < cuts_ref[g + 1])
    mw = jnp.where(mask, ws_ref[...], 0.0)
    out_ref[...] += mw * y


def _grouped_ffn(cuts, blk_ids, exp_ids, isf, isl, x_sorted, W1, b1r, W2, b2r, ws2d):
    grid_spec = pltpu.PrefetchScalarGridSpec(
        num_scalar_prefetch=5,
        grid=(G,),
        in_specs=[
            pl.BlockSpec((TM, H), lambda g, c, b, e, f, l: (b[g], 0)),
            pl.BlockSpec((1, H, I), lambda g, c, b, e, f, l: (e[g], 0, 0)),
            pl.BlockSpec((1, 1, I), lambda g, c, b, e, f, l: (e[g], 0, 0)),
            pl.BlockSpec((1, I, H), lambda g, c, b, e, f, l: (e[g], 0, 0)),
            pl.BlockSpec((1, 1, H), lambda g, c, b, e, f, l: (e[g], 0, 0)),
            pl.BlockSpec((TM, 1), lambda g, c, b, e, f, l: (b[g], 0)),
        ],
        out_specs=pl.BlockSpec((TM, H), lambda g, c, b, e, f, l: (b[g], 0)),
    )
    return pl.pallas_call(
        _ffn_body,
        grid_spec=grid_spec,
        out_shape=jax.ShapeDtypeStruct((N, H), jnp.float32),
    )(cuts, blk_ids, exp_ids, isf, isl, x_sorted, W1, b1r, W2, b2r, ws2d)


def kernel(hidden_states, Wg, W1, b1, W2, b2):
    x = hidden_states.reshape(S, H)
    wg_padded = jnp.pad(Wg, ((0, 0), (0, LANES - E)))

    idx_out, w_out, counts_out, loss_out = _router(x, wg_padded)
    balance_loss = loss_out[0, 0]
    counts = counts_out[0, :E]
    experts_flat = idx_out[:, :K].reshape(-1)
    weights_flat = w_out[:, :K].reshape(-1)

    offs = jnp.concatenate([jnp.zeros((1,), jnp.int32),
                            jnp.cumsum(counts, dtype=jnp.int32)])
    sort_idx = jnp.argsort(experts_flat).astype(jnp.int32)
    x_sorted = jnp.take(x, sort_idx // K, axis=0)
    ws2d = jnp.take(weights_flat, sort_idx)[:, None]

    blk_bounds = jnp.arange(NB, dtype=jnp.int32) * TM
    cuts = jnp.sort(jnp.concatenate([blk_bounds, offs[1:E]]))
    cuts_full = jnp.concatenate([cuts, jnp.full((1,), N, jnp.int32)])
    blk_ids = jnp.clip(cuts // TM, 0, NB - 1).astype(jnp.int32)
    exp_ids = jnp.clip(jnp.searchsorted(offs, cuts, side="right") - 1,
                       0, E - 1).astype(jnp.int32)
    prev = jnp.concatenate([jnp.full((1,), -1, jnp.int32), blk_ids[:-1]])
    nxt = jnp.concatenate([blk_ids[1:], jnp.full((1,), -1, jnp.int32)])
    isf = (blk_ids != prev).astype(jnp.int32)
    isl = (blk_ids != nxt).astype(jnp.int32)

    b1r = b1[:, None, :]
    b2r = b2[:, None, :]
    y_sorted = _grouped_ffn(cuts_full, blk_ids, exp_ids, isf, isl,
                            x_sorted, W1, b1r, W2, b2r, ws2d)

    inv = jnp.argsort(sort_idx).astype(jnp.int32)
    y_pairs = jnp.take(y_sorted, inv, axis=0).reshape(S, K, H)
    out = (x + y_pairs.sum(axis=1)).reshape(B, S, H)
    return out, balance_loss


# speedup vs baseline: 1.3493x; 1.1903x over previous
import functools

import functools

import jax
import jax.numpy as jnp
from jax import lax
from jax.experimental import pallas as pl
from jax.experimental.pallas import tpu as pltpu
from jax.experimental.pallas import tpu_sc as plsc

B, S, H, E, K, I = 1, 2048, 768, 8, 2, 3072
BALANCE_COEF = 0.01
N = B * S * K
TM = 512
NB = N // TM
G = NB + E - 1
LANES = 128


def _router_body(x_ref, wg_ref, idx_ref, w_ref, counts_ref, loss_ref):
    x = x_ref[...]
    wg = wg_ref[...]
    logits = jax.lax.dot_general(
        x, wg, (((1,), (0,)), ((), ())), preferred_element_type=jnp.float32)
    lane = jax.lax.broadcasted_iota(jnp.int32, (S, LANES), 1)
    valid = lane < E
    lg = jnp.where(valid, logits, -1e30)
    m = jnp.max(lg, axis=1, keepdims=True)
    p = jnp.where(valid, jnp.exp(lg - m), 0.0)
    probs = p / jnp.sum(p, axis=1, keepdims=True)
    v1 = jnp.max(probs, axis=1, keepdims=True)
    i1 = jnp.min(jnp.where(probs == v1, lane, LANES), axis=1, keepdims=True)
    probs_m = jnp.where(lane == i1, -1.0, probs)
    v2 = jnp.max(probs_m, axis=1, keepdims=True)
    i2 = jnp.min(jnp.where(probs_m == v2, lane, LANES), axis=1, keepdims=True)
    denom = v1 + v2
    idx_ref[...] = jnp.where(lane == 0, i1,
                             jnp.where(lane == 1, i2, 0)).astype(jnp.int32)
    w_ref[...] = jnp.where(lane == 0, v1 / denom,
                           jnp.where(lane == 1, v2 / denom, 0.0))
    onehot = ((lane == i1) | (lane == i2)).astype(jnp.float32)
    counts = jnp.sum(onehot, axis=0, keepdims=True)
    counts_ref[...] = counts.astype(jnp.int32)
    pmean = jnp.mean(probs, axis=0, keepdims=True)
    f = counts / jnp.float32(S)
    loss = BALANCE_COEF * E * jnp.sum(f * pmean)
    lane0 = jax.lax.broadcasted_iota(jnp.int32, (1, LANES), 1)
    loss_ref[...] = jnp.where(lane0 == 0, loss, 0.0)


def _router(x, wg_padded):
    return pl.pallas_call(
        _router_body,
        out_shape=(
            jax.ShapeDtypeStruct((S, LANES), jnp.int32),
            jax.ShapeDtypeStruct((S, LANES), jnp.float32),
            jax.ShapeDtypeStruct((1, LANES), jnp.int32),
            jax.ShapeDtypeStruct((1, LANES), jnp.float32),
        ),
    )(x, wg_padded)


def _ffn_body(cuts_ref, blk_ref, exp_ref, isf_ref, isl_ref,
              x_ref, w1_ref, b1_ref, w2_ref, b2_ref, ws_ref, out_ref):
    g = pl.program_id(0)

    @pl.when(isf_ref[g] == 1)
    def _():
        out_ref[...] = jnp.zeros_like(out_ref)

    x = x_ref[...].astype(jnp.bfloat16)
    h = jnp.dot(x, w1_ref[0].astype(jnp.bfloat16),
                preferred_element_type=jnp.float32) + b1_ref[0]
    h = jax.nn.gelu(h).astype(jnp.bfloat16)
    y = jnp.dot(h, w2_ref[0].astype(jnp.bfloat16),
                preferred_element_type=jnp.float32) + b2_ref[0]
    row = blk_ref[g] * TM + jax.lax.broadcasted_iota(jnp.int32, (TM, 1), 0)
    mask = (row >= cuts_ref[g]) & (row < cuts_ref[g + 1])
    mw = jnp.where(mask, ws_ref[...], 0.0)
    out_ref[...] += mw * y


def _grouped_ffn(cuts, blk_ids, exp_ids, isf, isl, x_sorted, W1, b1r, W2, b2r, ws2d):
    grid_spec = pltpu.PrefetchScalarGridSpec(
        num_scalar_prefetch=5,
        grid=(G,),
        in_specs=[
            pl.BlockSpec((TM, H), lambda g, c, b, e, f, l: (b[g], 0)),
            pl.BlockSpec((1, H, I), lambda g, c, b, e, f, l: (e[g], 0, 0)),
            pl.BlockSpec((1, 1, I), lambda g, c, b, e, f, l: (e[g], 0, 0)),
            pl.BlockSpec((1, I, H), lambda g, c, b, e, f, l: (e[g], 0, 0)),
            pl.BlockSpec((1, 1, H), lambda g, c, b, e, f, l: (e[g], 0, 0)),
            pl.BlockSpec((TM, 1), lambda g, c, b, e, f, l: (b[g], 0)),
        ],
        out_specs=pl.BlockSpec((TM, H), lambda g, c, b, e, f, l: (b[g], 0)),
    )
    return pl.pallas_call(
        _ffn_body,
        grid_spec=grid_spec,
        out_shape=jax.ShapeDtypeStruct((N, H), jnp.float32),
    )(cuts, blk_ids, exp_ids, isf, isl, x_sorted, W1, b1r, W2, b2r, ws2d)


_SC_NC, _SC_NS = 2, 16
_SC_NW = _SC_NC * _SC_NS
_TOK_W = S // _SC_NW
_TOK_C = 32
_NCHUNK = _TOK_W // _TOK_C


def _sc_combine_body(y_hbm, inv_hbm, x_hbm, out_hbm, idx_v, rows_v, x_v, o_v, sem):
    wid = lax.axis_index("s") * _SC_NC + lax.axis_index("c")
    for c in range(_NCHUNK):
        base = wid * _TOK_W + c * _TOK_C
        pltpu.sync_copy(inv_hbm.at[pl.ds(base * K, _TOK_C * K)], idx_v)
        pltpu.async_copy(y_hbm.at[idx_v], rows_v, sem).wait()
        pltpu.sync_copy(x_hbm.at[pl.ds(base, _TOK_C)], x_v)

        def row_body(r, carry):
            for j in range(H // 16):
                sl = pl.ds(j * 16, 16)
                o_v[r, sl] = x_v[r, sl] + rows_v[2 * r, sl] + rows_v[2 * r + 1, sl]
            return carry

        lax.fori_loop(0, _TOK_C, row_body, 0)
        pltpu.sync_copy(o_v, out_hbm.at[pl.ds(base, _TOK_C)])


def _sc_combine(y_sorted, inv, x):
    mesh = plsc.VectorSubcoreMesh(core_axis_name="c", subcore_axis_name="s")
    kfn = functools.partial(
        pl.kernel, mesh=mesh,
        out_type=jax.ShapeDtypeStruct((S, H), jnp.float32),
        scratch_types=[
            pltpu.VMEM((_TOK_C * K,), jnp.int32),
            pltpu.VMEM((_TOK_C * K, H), jnp.float32),
            pltpu.VMEM((_TOK_C, H), jnp.float32),
            pltpu.VMEM((_TOK_C, H), jnp.float32),
            pltpu.SemaphoreType.DMA,
        ],
    )(_sc_combine_body)
    return kfn(y_sorted, inv, x)


def kernel(hidden_states, Wg, W1, b1, W2, b2):
    x = hidden_states.reshape(S, H)
    wg_padded = jnp.pad(Wg, ((0, 0), (0, LANES - E)))

    idx_out, w_out, counts_out, loss_out = _router(x, wg_padded)
    balance_loss = loss_out[0, 0]
    counts = counts_out[0, :E]
    experts_flat = idx_out[:, :K].reshape(-1)
    weights_flat = w_out[:, :K].reshape(-1)

    offs = jnp.concatenate([jnp.zeros((1,), jnp.int32),
                            jnp.cumsum(counts, dtype=jnp.int32)])
    sort_idx = jnp.argsort(experts_flat).astype(jnp.int32)
    x_sorted = jnp.take(x, sort_idx // K, axis=0)
    ws2d = jnp.take(weights_flat, sort_idx)[:, None]

    blk_bounds = jnp.arange(NB, dtype=jnp.int32) * TM
    cuts = jnp.sort(jnp.concatenate([blk_bounds, offs[1:E]]))
    cuts_full = jnp.concatenate([cuts, jnp.full((1,), N, jnp.int32)])
    blk_ids = jnp.clip(cuts // TM, 0, NB - 1).astype(jnp.int32)
    exp_ids = jnp.clip(jnp.searchsorted(offs, cuts, side="right") - 1,
                       0, E - 1).astype(jnp.int32)
    prev = jnp.concatenate([jnp.full((1,), -1, jnp.int32), blk_ids[:-1]])
    nxt = jnp.concatenate([blk_ids[1:], jnp.full((1,), -1, jnp.int32)])
    isf = (blk_ids != prev).astype(jnp.int32)
    isl = (blk_ids != nxt).astype(jnp.int32)

    b1r = b1[:, None, :]
    b2r = b2[:, None, :]
    y_sorted = _grouped_ffn(cuts_full, blk_ids, exp_ids, isf, isl,
                            x_sorted, W1, b1r, W2, b2r, ws2d)

    inv = jnp.argsort(sort_idx).astype(jnp.int32)
    out = _sc_combine(y_sorted, inv, x).reshape(B, S, H)
    return out, balance_loss


# speedup vs baseline: 1.5966x; 1.1833x over previous
import functools

import functools

import jax
import jax.numpy as jnp
from jax import lax
from jax.experimental import pallas as pl
from jax.experimental.pallas import tpu as pltpu
from jax.experimental.pallas import tpu_sc as plsc

B, S, H, E, K, I = 1, 2048, 768, 8, 2, 3072
BALANCE_COEF = 0.01
N = B * S * K
TM = 512
NB = N // TM
G = NB + E - 1
LANES = 128


def _router_body(x_ref, wg_ref, idx_ref, w_ref, counts_ref, loss_ref):
    x = x_ref[...]
    wg = wg_ref[...]
    logits = jax.lax.dot_general(
        x, wg, (((1,), (0,)), ((), ())), preferred_element_type=jnp.float32)
    lane = jax.lax.broadcasted_iota(jnp.int32, (S, LANES), 1)
    valid = lane < E
    lg = jnp.where(valid, logits, -1e30)
    m = jnp.max(lg, axis=1, keepdims=True)
    p = jnp.where(valid, jnp.exp(lg - m), 0.0)
    probs = p / jnp.sum(p, axis=1, keepdims=True)
    v1 = jnp.max(probs, axis=1, keepdims=True)
    i1 = jnp.min(jnp.where(probs == v1, lane, LANES), axis=1, keepdims=True)
    probs_m = jnp.where(lane == i1, -1.0, probs)
    v2 = jnp.max(probs_m, axis=1, keepdims=True)
    i2 = jnp.min(jnp.where(probs_m == v2, lane, LANES), axis=1, keepdims=True)
    denom = v1 + v2
    idx_ref[...] = jnp.where(lane == 0, i1,
                             jnp.where(lane == 1, i2, 0)).astype(jnp.int32)
    w_ref[...] = jnp.where(lane == 0, v1 / denom,
                           jnp.where(lane == 1, v2 / denom, 0.0))
    onehot = ((lane == i1) | (lane == i2)).astype(jnp.float32)
    counts = jnp.sum(onehot, axis=0, keepdims=True)
    counts_ref[...] = counts.astype(jnp.int32)
    pmean = jnp.mean(probs, axis=0, keepdims=True)
    f = counts / jnp.float32(S)
    loss = BALANCE_COEF * E * jnp.sum(f * pmean)
    lane0 = jax.lax.broadcasted_iota(jnp.int32, (1, LANES), 1)
    loss_ref[...] = jnp.where(lane0 == 0, loss, 0.0)


def _router(x, wg_padded):
    return pl.pallas_call(
        _router_body,
        out_shape=(
            jax.ShapeDtypeStruct((S, LANES), jnp.int32),
            jax.ShapeDtypeStruct((S, LANES), jnp.float32),
            jax.ShapeDtypeStruct((1, LANES), jnp.int32),
            jax.ShapeDtypeStruct((1, LANES), jnp.float32),
        ),
    )(x, wg_padded)


def _ffn_body(cuts_ref, blk_ref, exp_ref, isf_ref, isl_ref,
              x_ref, w1_ref, b1_ref, w2_ref, b2_ref, ws_ref, out_ref):
    g = pl.program_id(0)

    @pl.when(isf_ref[g] == 1)
    def _():
        out_ref[...] = jnp.zeros_like(out_ref)

    x = x_ref[...].astype(jnp.bfloat16)
    h = jnp.dot(x, w1_ref[0].astype(jnp.bfloat16),
                preferred_element_type=jnp.float32) + b1_ref[0]
    h = jax.nn.gelu(h).astype(jnp.bfloat16)
    y = jnp.dot(h, w2_ref[0].astype(jnp.bfloat16),
                preferred_element_type=jnp.float32) + b2_ref[0]
    row = blk_ref[g] * TM + jax.lax.broadcasted_iota(jnp.int32, (TM, 1), 0)
    mask = (row >= cuts_ref[g]) & (row < cuts_ref[g + 1])
    mw = jnp.where(mask, ws_ref[...], 0.0)
    out_ref[...] += mw * y


def _grouped_ffn(cuts, blk_ids, exp_ids, isf, isl, x_sorted, W1, b1r, W2, b2r, ws2d):
    grid_spec = pltpu.PrefetchScalarGridSpec(
        num_scalar_prefetch=5,
        grid=(G,),
        in_specs=[
            pl.BlockSpec((TM, H), lambda g, c, b, e, f, l: (b[g], 0)),
            pl.BlockSpec((1, H, I), lambda g, c, b, e, f, l: (e[g], 0, 0)),
            pl.BlockSpec((1, 1, I), lambda g, c, b, e, f, l: (e[g], 0, 0)),
            pl.BlockSpec((1, I, H), lambda g, c, b, e, f, l: (e[g], 0, 0)),
            pl.BlockSpec((1, 1, H), lambda g, c, b, e, f, l: (e[g], 0, 0)),
            pl.BlockSpec((TM, 1), lambda g, c, b, e, f, l: (b[g], 0)),
        ],
        out_specs=pl.BlockSpec((TM, H), lambda g, c, b, e, f, l: (b[g], 0)),
    )
    return pl.pallas_call(
        _ffn_body,
        grid_spec=grid_spec,
        out_shape=jax.ShapeDtypeStruct((N, H), jnp.float32),
    )(cuts, blk_ids, exp_ids, isf, isl, x_sorted, W1, b1r, W2, b2r, ws2d)


_SC_NC, _SC_NS = 2, 16
_SC_NW = _SC_NC * _SC_NS
_TOK_W = S // _SC_NW
_TOK_C = 32
_NCHUNK = _TOK_W // _TOK_C


def _sc_combine_body(y_hbm, inv_hbm, x_hbm, out_hbm, idx_v, rows_v, x_v, o_v, sem):
    wid = lax.axis_index("s") * _SC_NC + lax.axis_index("c")
    for c in range(_NCHUNK):
        base = wid * _TOK_W + c * _TOK_C
        pltpu.sync_copy(inv_hbm.at[pl.ds(base * K, _TOK_C * K)], idx_v)
        pltpu.async_copy(y_hbm.at[idx_v], rows_v, sem).wait()
        pltpu.sync_copy(x_hbm.at[pl.ds(base, _TOK_C)], x_v)

        def row_body(r, carry):
            for j in range(H // 16):
                sl = pl.ds(j * 16, 16)
                o_v[r, sl] = x_v[r, sl] + rows_v[2 * r, sl] + rows_v[2 * r + 1, sl]
            return carry

        lax.fori_loop(0, _TOK_C, row_body, 0)
        pltpu.sync_copy(o_v, out_hbm.at[pl.ds(base, _TOK_C)])


def _sc_dispatch_body(x_hbm, tok_hbm, out_hbm, idx_v, rows_v, sem):
    wid = lax.axis_index("s") * _SC_NC + lax.axis_index("c")
    rows_per_w = N // _SC_NW
    for c in range(rows_per_w // (_TOK_C * K)):
        base = wid * rows_per_w + c * _TOK_C * K
        pltpu.sync_copy(tok_hbm.at[pl.ds(base, _TOK_C * K)], idx_v)
        pltpu.async_copy(x_hbm.at[idx_v], rows_v, sem).wait()
        pltpu.sync_copy(rows_v, out_hbm.at[pl.ds(base, _TOK_C * K)])


def _sc_dispatch(x, tok_ids):
    mesh = plsc.VectorSubcoreMesh(core_axis_name="c", subcore_axis_name="s")
    kfn = functools.partial(
        pl.kernel, mesh=mesh,
        out_type=jax.ShapeDtypeStruct((N, H), jnp.float32),
        scratch_types=[
            pltpu.VMEM((_TOK_C * K,), jnp.int32),
            pltpu.VMEM((_TOK_C * K, H), jnp.float32),
            pltpu.SemaphoreType.DMA,
        ],
    )(_sc_dispatch_body)
    return kfn(x, tok_ids)


def _sc_combine(y_sorted, inv, x):
    mesh = plsc.VectorSubcoreMesh(core_axis_name="c", subcore_axis_name="s")
    kfn = functools.partial(
        pl.kernel, mesh=mesh,
        out_type=jax.ShapeDtypeStruct((S, H), jnp.float32),
        scratch_types=[
            pltpu.VMEM((_TOK_C * K,), jnp.int32),
            pltpu.VMEM((_TOK_C * K, H), jnp.float32),
            pltpu.VMEM((_TOK_C, H), jnp.float32),
            pltpu.VMEM((_TOK_C, H), jnp.float32),
            pltpu.SemaphoreType.DMA,
        ],
    )(_sc_combine_body)
    return kfn(y_sorted, inv, x)


def kernel(hidden_states, Wg, W1, b1, W2, b2):
    x = hidden_states.reshape(S, H)
    wg_padded = jnp.pad(Wg, ((0, 0), (0, LANES - E)))

    idx_out, w_out, counts_out, loss_out = _router(x, wg_padded)
    balance_loss = loss_out[0, 0]
    counts = counts_out[0, :E]
    experts_flat = idx_out[:, :K].reshape(-1)
    weights_flat = w_out[:, :K].reshape(-1)

    offs = jnp.concatenate([jnp.zeros((1,), jnp.int32),
                            jnp.cumsum(counts, dtype=jnp.int32)])
    sort_idx = jnp.argsort(experts_flat).astype(jnp.int32)
    x_sorted = _sc_dispatch(x, sort_idx // K)
    ws2d = jnp.take(weights_flat, sort_idx)[:, None]

    blk_bounds = jnp.arange(NB, dtype=jnp.int32) * TM
    cuts = jnp.sort(jnp.concatenate([blk_bounds, offs[1:E]]))
    cuts_full = jnp.concatenate([cuts, jnp.full((1,), N, jnp.int32)])
    blk_ids = jnp.clip(cuts // TM, 0, NB - 1).astype(jnp.int32)
    exp_ids = jnp.clip(jnp.searchsorted(offs, cuts, side="right") - 1,
                       0, E - 1).astype(jnp.int32)
    prev = jnp.concatenate([jnp.full((1,), -1, jnp.int32), blk_ids[:-1]])
    nxt = jnp.concatenate([blk_ids[1:], jnp.full((1,), -1, jnp.int32)])
    isf = (blk_ids != prev).astype(jnp.int32)
    isl = (blk_ids != nxt).astype(jnp.int32)

    b1r = b1[:, None, :]
    b2r = b2[:, None, :]
    y_sorted = _grouped_ffn(cuts_full, blk_ids, exp_ids, isf, isl,
                            x_sorted, W1, b1r, W2, b2r, ws2d)

    inv = jnp.argsort(sort_idx).astype(jnp.int32)
    out = _sc_combine(y_sorted, inv, x).reshape(B, S, H)
    return out, balance_loss
